# baseline (device time: 183297 ns/iter reference)
import jax
import jax.numpy as jnp
from jax import lax
from jax.experimental import pallas as pl
from jax.experimental.pallas import tpu as pltpu

N_DEV = 4


def kernel(x, w_mat, scale_x, scale_w):
    m, k_shard = x.shape
    _, n = w_mat.shape
    kh = k_shard // 2
    mc = m // N_DEV

    x8 = x.astype(jnp.float8_e4m3fn)
    w8 = w_mat.astype(jnp.float8_e4m3fn)

    def body(x_ref, w_ref, sx_ref, sw_ref, out_ref,
             xf, wf, acc, tmp,
             xs_r, ws_r, xs_l, ws_l,
             xr_r, wr_r, xr_l, wr_l,
             dma_sem):
        my = lax.axis_index("i")
        left = (my + N_DEV - 1) % N_DEV
        right = (my + 1) % N_DEV

        def c(k):
            return (my + 8 * N_DEV + k) % N_DEV

        barrier_sem = pltpu.get_barrier_semaphore()
        for nbr in [left, right]:
            pl.semaphore_signal(
                barrier_sem, inc=1,
                device_id=(nbr,), device_id_type=pl.DeviceIdType.MESH,
            )
        pl.semaphore_wait(barrier_sem, 2)

        pending = []

        def send(src, dst, ssem, rsem, to):
            rdma = pltpu.make_async_remote_copy(
                src_ref=src, dst_ref=dst, send_sem=ssem, recv_sem=rsem,
                device_id=(to,), device_id_type=pl.DeviceIdType.MESH,
            )
            rdma.start()
            pending.append(rdma)
            return rdma

        def accum2(jr, jl, init):
            for r in range(N_DEV):
                d = jnp.dot(
                    xf[jr, 0, pl.ds(r * mc, mc), :], wf[jr, 0],
                    preferred_element_type=jnp.float32,
                ) + jnp.dot(
                    xf[jl, 1, pl.ds(r * mc, mc), :], wf[jl, 1],
                    preferred_element_type=jnp.float32,
                )
                if init:
                    acc[pl.ds(r * mc, mc), :] = d.astype(jnp.bfloat16)
                else:
                    acc[pl.ds(r * mc, mc), :] = (
                        acc[pl.ds(r * mc, mc), :].astype(jnp.float32) + d
                    ).astype(jnp.bfloat16)

        recvs = []
        for s in range(N_DEV - 1):
            if s == 0:
                sr_x, sr_w = x_ref.at[:, :kh], w_ref.at[pl.ds(0, kh), :]
                sl_x, sl_w = x_ref.at[:, kh:], w_ref.at[pl.ds(kh, kh), :]
            else:
                sr_x, sr_w = xf.at[c(-s), 0], wf.at[c(-s), 0]
                sl_x, sl_w = xf.at[c(s), 1], wf.at[c(s), 1]
            step = [
                send(sr_x, xf.at[c(-s), 0], xs_r.at[s], xr_r.at[s], right),
                send(sr_w, wf.at[c(-s), 0], ws_r.at[s], wr_r.at[s], right),
                send(sl_x, xf.at[c(s), 1], xs_l.at[s], xr_l.at[s], left),
                send(sl_w, wf.at[c(s), 1], ws_l.at[s], wr_l.at[s], left),
            ]
            if s == 0:
                for r in range(N_DEV):
                    acc[pl.ds(r * mc, mc), :] = (
                        jnp.dot(
                            x_ref[pl.ds(r * mc, mc), :kh],
                            w_ref[pl.ds(0, kh), :],
                            preferred_element_type=jnp.float32,
                        )
                        + jnp.dot(
                            x_ref[pl.ds(r * mc, mc), kh:],
                            w_ref[pl.ds(kh, kh), :],
                            preferred_element_type=jnp.float32,
                        )
                    ).astype(jnp.bfloat16)
            else:
                accum2(c(-s), c(s), False)
            for rdma in step:
                rdma.wait_recv()
            recvs.append(step)

        scale = sx_ref[0] * sw_ref[0]
        prev_cp = None
        for r in range(N_DEV):
            t = (
                acc[pl.ds(r * mc, mc), :].astype(jnp.float32)
                + jnp.dot(
                    xf[c(1), 0, pl.ds(r * mc, mc), :], wf[c(1), 0],
                    preferred_element_type=jnp.float32,
                )
                + jnp.dot(
                    xf[c(-1), 1, pl.ds(r * mc, mc), :], wf[c(-1), 1],
                    preferred_element_type=jnp.float32,
                )
            )
            y = t * scale
            z = y * (1.0 / (1.0 + jnp.exp(-y)))
            if prev_cp is not None:
                prev_cp.wait()
            tmp[...] = z
            prev_cp = pltpu.make_async_copy(
                tmp, out_ref.at[pl.ds(r * mc, mc), :], dma_sem
            )
            prev_cp.start()
        prev_cp.wait()

        for rdma in pending:
            rdma.wait_send()

    out_shape = jax.ShapeDtypeStruct((m, n), jnp.float32)
    return pl.pallas_call(
        body,
        out_shape=out_shape,
        in_specs=[
            pl.BlockSpec(memory_space=pltpu.VMEM),
            pl.BlockSpec(memory_space=pltpu.VMEM),
            pl.BlockSpec(memory_space=pltpu.SMEM),
            pl.BlockSpec(memory_space=pltpu.SMEM),
        ],
        out_specs=pl.BlockSpec(memory_space=pltpu.MemorySpace.HBM),
        scratch_shapes=[
            pltpu.VMEM((N_DEV, 2, m, kh), jnp.float8_e4m3fn),
            pltpu.VMEM((N_DEV, 2, kh, n), jnp.float8_e4m3fn),
            pltpu.VMEM((m, n), jnp.bfloat16),
            pltpu.VMEM((mc, n), jnp.float32),
            pltpu.SemaphoreType.DMA((N_DEV - 1,)),
            pltpu.SemaphoreType.DMA((N_DEV - 1,)),
            pltpu.SemaphoreType.DMA((N_DEV - 1,)),
            pltpu.SemaphoreType.DMA((N_DEV - 1,)),
            pltpu.SemaphoreType.DMA((N_DEV - 1,)),
            pltpu.SemaphoreType.DMA((N_DEV - 1,)),
            pltpu.SemaphoreType.DMA((N_DEV - 1,)),
            pltpu.SemaphoreType.DMA((N_DEV - 1,)),
            pltpu.SemaphoreType.DMA,
        ],
        compiler_params=pltpu.CompilerParams(
            collective_id=0,
            vmem_limit_bytes=62 * 1024 * 1024,
        ),
    )(x8, w8, scale_x, scale_w)
